# baseline (speedup 1.0000x reference)
import functools

import jax
import jax.numpy as jnp
from jax import lax
from jax.experimental import pallas as pl
from jax.experimental.pallas import tpu as pltpu
from jax.experimental.pallas import tpu_sc as plsc

N = 10000
E = 320000
D = 128
NC, NS = 2, 16
NW = NC * NS
EPW = E // NW
B = 80
BLOCKS = EPW // B
ROWS = E // B
NPAD = 10240
TPN = NPAD // NS
PITER = (BLOCKS + 2) // 3
AGGP = 10112
TP3 = AGGP // NS

_f32 = jnp.float32
_i32 = jnp.int32
_mesh = plsc.VectorSubcoreMesh(
    core_axis_name="c", subcore_axis_name="s", num_cores=NC, num_subcores=NS)


def _k1_body(x_ref, ws_ref, wd_ref, xs_ref, xd_ref):
    xv = x_ref[...]
    xs_ref[...] = jnp.dot(xv, ws_ref[...], preferred_element_type=_f32)
    xd_ref[...] = jnp.dot(xv, wd_ref[...], preferred_element_type=_f32)


_k1 = pl.pallas_call(
    _k1_body,
    grid=(10,),
    in_specs=[
        pl.BlockSpec((N // 10, D), lambda i: (i, 0)),
        pl.BlockSpec((D, D), lambda i: (0, 0)),
        pl.BlockSpec((D, D), lambda i: (0, 0)),
    ],
    out_specs=[pl.BlockSpec((N // 10, D), lambda i: (i, 0))] * 2,
    out_shape=[jax.ShapeDtypeStruct((N, D), _f32)] * 2,
)


@functools.partial(
    pl.kernel,
    out_type=[
        jax.ShapeDtypeStruct((E, D), _f32),
        jax.ShapeDtypeStruct((E,), _f32),
        jax.ShapeDtypeStruct((NC, NPAD), _f32),
    ],
    mesh=_mesh,
    scratch_types=[
        pltpu.VMEM((BLOCKS, B), _i32),
        pltpu.VMEM((BLOCKS, B), _i32),
        pltpu.VMEM((B, D), _f32),
        pltpu.VMEM((B, D), _f32),
        pltpu.VMEM((B, D), _f32),
        pltpu.VMEM((B, D), _f32),
        pltpu.VMEM((B, D), _f32),
        pltpu.VMEM((B, D), _f32),
        pltpu.VMEM((B,), _f32),
        pltpu.VMEM((B,), _f32),
        pltpu.VMEM((B,), _f32),
        pltpu.VMEM((D,), _f32),
        pltpu.VMEM((D,), _f32),
        pltpu.VMEM((TPN,), _f32),
        pltpu.VMEM_SHARED((NPAD,), _f32),
        pltpu.SemaphoreType.DMA,
        pltpu.SemaphoreType.DMA,
        pltpu.SemaphoreType.DMA,
        pltpu.SemaphoreType.DMA,
        pltpu.SemaphoreType.DMA,
        pltpu.SemaphoreType.DMA,
        pltpu.SemaphoreType.DMA,
        pltpu.SemaphoreType.DMA,
        pltpu.SemaphoreType.DMA,
    ],
)
def _k2(xs_hbm, xd_hbm, src_hbm, dst_hbm, b_hbm, a_hbm,
        h_out, e_out, den_out,
        src_all, dst_all, xs_r0, xs_r1, xs_r2, xd_r0, xd_r1, xd_r2,
        e_v0, e_v1, e_v2, b_v, a_v, zb, den_sh,
        gs0, gs1, gs2, ws0, ws1, ws2, ds0, ds1, ds2):
    cid = lax.axis_index("c")
    sid = lax.axis_index("s")
    wid = sid * NC + cid
    base = wid * EPW

    xsr = [xs_r0, xs_r1, xs_r2]
    xdr = [xd_r0, xd_r1, xd_r2]
    ev = [e_v0, e_v1, e_v2]
    gsem = [gs0, gs1, gs2]
    wsem = [ws0, ws1, ws2]
    dsem = [ds0, ds1, ds2]

    pltpu.sync_copy(b_hbm, b_v)
    pltpu.sync_copy(a_hbm, a_v)
    pltpu.sync_copy(src_hbm.at[wid], src_all)
    pltpu.sync_copy(dst_hbm.at[wid], dst_all)

    b_ch = [b_v[pl.ds(j * 16, 16)] for j in range(D // 16)]
    a_ch = [a_v[pl.ds(j * 16, 16)] for j in range(D // 16)]
    lane = lax.iota(_i32, 16)
    bfly = [lane ^ m for m in (8, 4, 2, 1)]

    @pl.loop(0, TPN // 16)
    def _z(i):
        zb[pl.ds(i * 16, 16)] = jnp.zeros((16,), _f32)

    pltpu.sync_copy(zb, den_sh.at[pl.ds(sid * TPN, TPN)])
    plsc.subcore_barrier()

    def fire_g(i, k):
        pltpu.async_copy(xs_hbm.at[src_all.at[i]], xsr[k], gsem[k])
        pltpu.async_copy(xd_hbm.at[dst_all.at[i]], xdr[k], gsem[k])

    def wait_g(i, k):
        pltpu.make_async_copy(xs_hbm.at[src_all.at[i]], xsr[k], gsem[k]).wait()
        pltpu.make_async_copy(xd_hbm.at[dst_all.at[i]], xdr[k], gsem[k]).wait()

    def fire_w(i, k):
        off = base + i * B
        pltpu.async_copy(xsr[k], h_out.at[pl.ds(off, B)], wsem[k])
        pltpu.async_copy(ev[k], e_out.at[pl.ds(off, B)], wsem[k])

    def wait_w(i, k):
        off = base + i * B
        pltpu.make_async_copy(xsr[k], h_out.at[pl.ds(off, B)], wsem[k]).wait()
        pltpu.make_async_copy(ev[k], e_out.at[pl.ds(off, B)], wsem[k]).wait()

    def fire_d(i, k):
        pltpu.async_copy(ev[k], den_sh.at[dst_all.at[i]], dsem[k], add=True)

    def wait_d(i, k):
        pltpu.make_async_copy(ev[k], den_sh.at[dst_all.at[i]], dsem[k]).wait()

    def compute(i, k):
        xr, dr, e_b = xsr[k], xdr[k], ev[k]

        @pl.loop(0, B // 16)
        def _grp(t):
            sv = jnp.zeros((16,), _f32)
            for kk in range(16):
                e = t * 16 + kk
                acc = jnp.zeros((16,), _f32)
                for j in range(D // 16):
                    sl = pl.ds(j * 16, 16)
                    g = xr[e, sl] + dr[e, sl] + b_ch[j]
                    hv = jnp.maximum(g, 0.2 * g)
                    xr[e, sl] = hv
                    acc = acc + hv * a_ch[j]
                for p_ in bfly:
                    acc = acc + jnp.take_along_axis(acc, p_, axis=0)
                sv = jnp.where(lane == kk, acc, sv)
            e_b[pl.ds(t * 16, 16)] = jnp.exp(sv)

    fire_g(0, 0)
    fire_g(1, 1)

    @pl.loop(0, PITER)
    def _pipe(p):
        i0 = 3 * p
        wait_g(i0, 0)
        compute(i0, 0)
        fire_d(i0, 0)
        fire_w(i0, 0)

        @pl.when(p > 0)
        def _():
            wait_w(i0 - 1, 2)
            wait_d(i0 - 1, 2)

        @pl.when(p < PITER - 1)
        def _():
            fire_g(i0 + 2, 2)

        i1 = 3 * p + 1
        wait_g(i1, 1)
        compute(i1, 1)
        fire_d(i1, 1)
        fire_w(i1, 1)
        wait_w(i1 - 1, 0)
        wait_d(i1 - 1, 0)

        @pl.when(p < PITER - 1)
        def _():
            fire_g(i1 + 2, 0)

        @pl.when(p < PITER - 1)
        def _():
            i2 = 3 * p + 2
            wait_g(i2, 2)
            compute(i2, 2)
            fire_d(i2, 2)
            fire_w(i2, 2)
            wait_w(i2 - 1, 1)
            wait_d(i2 - 1, 1)
            fire_g(i2 + 2, 1)

    wait_w(BLOCKS - 1, 1)
    wait_d(BLOCKS - 1, 1)
    plsc.subcore_barrier()
    pltpu.sync_copy(den_sh.at[pl.ds(sid * TPN, TPN)],
                    den_out.at[cid, pl.ds(sid * TPN, TPN)])


def _k2b_body(den_ref, out_ref):
    d = den_ref[...]
    out_ref[...] = 1.0 / (d[0:1, :] + d[1:2, :] + 1e-16)


_k2b = pl.pallas_call(
    _k2b_body,
    in_specs=[pl.BlockSpec((NC, NPAD), lambda: (0, 0))],
    out_specs=pl.BlockSpec((1, NPAD), lambda: (0, 0)),
    out_shape=jax.ShapeDtypeStruct((1, NPAD), _f32),
)


@functools.partial(
    pl.kernel,
    out_type=jax.ShapeDtypeStruct((NC, AGGP, D), _f32),
    mesh=_mesh,
    scratch_types=[
        pltpu.VMEM((B, D), _f32),
        pltpu.VMEM((B, D), _f32),
        pltpu.VMEM((B, D), _f32),
        pltpu.VMEM((B,), _i32),
        pltpu.VMEM((B,), _i32),
        pltpu.VMEM((B,), _i32),
        pltpu.VMEM((B,), _i32),
        pltpu.VMEM((B,), _i32),
        pltpu.VMEM((B,), _i32),
        pltpu.VMEM((B,), _f32),
        pltpu.VMEM((B,), _f32),
        pltpu.VMEM((B,), _f32),
        pltpu.VMEM((B,), _f32),
        pltpu.VMEM((B,), _f32),
        pltpu.VMEM((B,), _f32),
        pltpu.VMEM_SHARED((AGGP, D), _f32),
        pltpu.SemaphoreType.DMA,
        pltpu.SemaphoreType.DMA,
        pltpu.SemaphoreType.DMA,
        pltpu.SemaphoreType.DMA,
        pltpu.SemaphoreType.DMA,
        pltpu.SemaphoreType.DMA,
        pltpu.SemaphoreType.DMA,
        pltpu.SemaphoreType.DMA,
        pltpu.SemaphoreType.DMA,
    ],
)
def _k3(h_hbm, e_hbm, dst_hbm, hh_hbm, dinv_hbm, z_hbm,
        agg_out,
        h_r0, h_r1, h_r2, dst_0, dst_1, dst_2, hh_0, hh_1, hh_2,
        e_0, e_1, e_2, d_0, d_1, d_2, agg_sh,
        as0, as1, as2, gs0, gs1, gs2, ss0, ss1, ss2):
    cid = lax.axis_index("c")
    sid = lax.axis_index("s")
    wid = sid * NC + cid
    base = wid * EPW

    hr = [h_r0, h_r1, h_r2]
    dstb = [dst_0, dst_1, dst_2]
    hhb = [hh_0, hh_1, hh_2]
    eb = [e_0, e_1, e_2]
    db = [d_0, d_1, d_2]
    asem = [as0, as1, as2]
    gsem = [gs0, gs1, gs2]
    ssem = [ss0, ss1, ss2]

    pltpu.sync_copy(z_hbm.at[pl.ds(0, TP3)], agg_sh.at[pl.ds(sid * TP3, TP3)])
    plsc.subcore_barrier()

    def fire_a(i, k):
        pltpu.async_copy(dst_hbm.at[wid, i], dstb[k], asem[k])
        pltpu.async_copy(e_hbm.at[wid, i], eb[k], asem[k])
        pltpu.async_copy(hh_hbm.at[wid, i], hhb[k], asem[k])

    def wait_a(i, k):
        pltpu.make_async_copy(dst_hbm.at[wid, i], dstb[k], asem[k]).wait()
        pltpu.make_async_copy(e_hbm.at[wid, i], eb[k], asem[k]).wait()
        pltpu.make_async_copy(hh_hbm.at[wid, i], hhb[k], asem[k]).wait()

    def fire_b(i, k):
        off = base + i * B
        pltpu.async_copy(h_hbm.at[pl.ds(off, B)], hr[k], gsem[k])
        pltpu.async_copy(dinv_hbm.at[dstb[k]], db[k], gsem[k])

    def wait_b(i, k):
        off = base + i * B
        pltpu.make_async_copy(h_hbm.at[pl.ds(off, B)], hr[k], gsem[k]).wait()
        pltpu.make_async_copy(dinv_hbm.at[dstb[k]], db[k], gsem[k]).wait()

    def fire_s(i, k):
        pltpu.async_copy(hr[k], agg_sh.at[hhb[k]], ssem[k], add=True)

    def wait_s(i, k):
        pltpu.make_async_copy(hr[k], agg_sh.at[hhb[k]], ssem[k]).wait()

    def compute(i, k):
        hb, e_b, d_b = hr[k], eb[k], db[k]

        @pl.loop(0, B // 16)
        def _grp(t):
            sl16 = pl.ds(t * 16, 16)
            al16 = e_b[sl16] * d_b[sl16]
            for kk in range(16):
                e = t * 16 + kk
                al = al16[kk]
                for j in range(D // 16):
                    sl = pl.ds(j * 16, 16)
                    hb[e, sl] = hb[e, sl] * al

    fire_a(0, 0)
    fire_a(1, 1)
    wait_a(0, 0)
    fire_b(0, 0)

    @pl.loop(0, PITER)
    def _pipe(p):
        i0 = 3 * p
        wait_a(i0 + 1, 1)
        fire_b(i0 + 1, 1)
        wait_b(i0, 0)
        compute(i0, 0)
        fire_s(i0, 0)

        @pl.when(p > 0)
        def _():
            wait_s(i0 - 1, 2)

        @pl.when(p < PITER - 1)
        def _():
            fire_a(i0 + 2, 2)

        i1 = 3 * p + 1

        @pl.when(p < PITER - 1)
        def _():
            wait_a(i1 + 1, 2)
            fire_b(i1 + 1, 2)

        wait_b(i1, 1)
        compute(i1, 1)
        fire_s(i1, 1)
        wait_s(i1 - 1, 0)

        @pl.when(p < PITER - 1)
        def _():
            fire_a(i1 + 2, 0)

        @pl.when(p < PITER - 1)
        def _():
            i2 = 3 * p + 2
            wait_a(i2 + 1, 0)
            fire_b(i2 + 1, 0)
            wait_b(i2, 2)
            compute(i2, 2)
            fire_s(i2, 2)
            wait_s(i2 - 1, 1)
            fire_a(i2 + 2, 1)

    wait_s(BLOCKS - 1, 1)
    plsc.subcore_barrier()
    pltpu.sync_copy(agg_sh.at[pl.ds(sid * TP3, TP3)],
                    agg_out.at[cid, pl.ds(sid * TP3, TP3)])


def _k4_body(agg_ref, we_ref, be_ref, wo_ref, out_ref):
    av = agg_ref[...]
    a = av[0] + av[1]
    t = jnp.dot(a, we_ref[...], preferred_element_type=_f32) + be_ref[...]
    t = jnp.maximum(t, 0.2 * t)
    out_ref[...] = jnp.dot(t, wo_ref[...], preferred_element_type=_f32)


_k4 = pl.pallas_call(
    _k4_body,
    grid=(10,),
    in_specs=[
        pl.BlockSpec((2, N // 10, D), lambda i: (0, i, 0)),
        pl.BlockSpec((D, D), lambda i: (0, 0)),
        pl.BlockSpec((1, D), lambda i: (0, 0)),
        pl.BlockSpec((D, D), lambda i: (0, 0)),
    ],
    out_specs=pl.BlockSpec((N // 10, D), lambda i: (i, 0)),
    out_shape=jax.ShapeDtypeStruct((N, D), _f32),
)


def kernel(x, edge_index, H, W_src, W_dst, b_tsa, a_vec, W_etn, b_etn, W_out):
    src2 = edge_index[0].astype(_i32).reshape(NW, BLOCKS, B)
    dst2 = edge_index[1].astype(_i32).reshape(NW, BLOCKS, B)
    hh2 = H.astype(_i32).reshape(NW, BLOCKS, B)
    xs, xd = _k1(x, W_src, W_dst)
    h, ev, den = _k2(xs, xd, src2, dst2, b_tsa, a_vec)
    dinv = _k2b(den).reshape(NPAD)
    zeros = jnp.zeros((TPN, D), _f32)
    agg2 = _k3(h, ev.reshape(NW, BLOCKS, B), dst2, hh2, dinv, zeros)
    return _k4(agg2, W_etn, b_etn.reshape(1, D), W_out)


# speedup vs baseline: 1.0604x; 1.0604x over previous
import functools

import jax
import jax.numpy as jnp
from jax import lax
from jax.experimental import pallas as pl
from jax.experimental.pallas import tpu as pltpu
from jax.experimental.pallas import tpu_sc as plsc

N = 10000
E = 320000
D = 128
NC, NS = 2, 16
NW = NC * NS
EPW = E // NW
B = 80
BLOCKS = EPW // B
ROWS = E // B
NPAD = 10240
TPN = NPAD // NS
PITER = (BLOCKS + 2) // 3
AGGP = 10112
TP3 = AGGP // NS

_f32 = jnp.float32
_i32 = jnp.int32
_mesh = plsc.VectorSubcoreMesh(
    core_axis_name="c", subcore_axis_name="s", num_cores=NC, num_subcores=NS)


def _k1_body(x_ref, ws_ref, wd_ref, xs_ref, xd_ref):
    xv = x_ref[...]
    xs_ref[...] = jnp.dot(xv, ws_ref[...], preferred_element_type=_f32)
    xd_ref[...] = jnp.dot(xv, wd_ref[...], preferred_element_type=_f32)


_k1 = pl.pallas_call(
    _k1_body,
    grid=(10,),
    in_specs=[
        pl.BlockSpec((N // 10, D), lambda i: (i, 0)),
        pl.BlockSpec((D, D), lambda i: (0, 0)),
        pl.BlockSpec((D, D), lambda i: (0, 0)),
    ],
    out_specs=[pl.BlockSpec((N // 10, D), lambda i: (i, 0))] * 2,
    out_shape=[jax.ShapeDtypeStruct((N, D), _f32)] * 2,
)


@functools.partial(
    pl.kernel,
    out_type=[
        jax.ShapeDtypeStruct((E, D), _f32),
        jax.ShapeDtypeStruct((E,), _f32),
        jax.ShapeDtypeStruct((NC, NPAD), _f32),
    ],
    mesh=_mesh,
    scratch_types=[
        pltpu.VMEM((BLOCKS, B), _i32),
        pltpu.VMEM((BLOCKS, B), _i32),
        pltpu.VMEM((B, D), _f32),
        pltpu.VMEM((B, D), _f32),
        pltpu.VMEM((B, D), _f32),
        pltpu.VMEM((B, D), _f32),
        pltpu.VMEM((B, D), _f32),
        pltpu.VMEM((B, D), _f32),
        pltpu.VMEM((B,), _f32),
        pltpu.VMEM((B,), _f32),
        pltpu.VMEM((B,), _f32),
        pltpu.VMEM((D,), _f32),
        pltpu.VMEM((D,), _f32),
        pltpu.VMEM((TPN,), _f32),
        pltpu.VMEM_SHARED((NPAD,), _f32),
        pltpu.SemaphoreType.DMA,
        pltpu.SemaphoreType.DMA,
        pltpu.SemaphoreType.DMA,
        pltpu.SemaphoreType.DMA,
        pltpu.SemaphoreType.DMA,
        pltpu.SemaphoreType.DMA,
        pltpu.SemaphoreType.DMA,
        pltpu.SemaphoreType.DMA,
        pltpu.SemaphoreType.DMA,
    ],
)
def _k2(xs_hbm, xd_hbm, src_hbm, dst_hbm, b_hbm, a_hbm,
        h_out, e_out, den_out,
        src_all, dst_all, xs_r0, xs_r1, xs_r2, xd_r0, xd_r1, xd_r2,
        e_v0, e_v1, e_v2, b_v, a_v, zb, den_sh,
        gs0, gs1, gs2, ws0, ws1, ws2, ds0, ds1, ds2):
    cid = lax.axis_index("c")
    sid = lax.axis_index("s")
    wid = sid * NC + cid
    base = wid * EPW

    xsr = [xs_r0, xs_r1, xs_r2]
    xdr = [xd_r0, xd_r1, xd_r2]
    ev = [e_v0, e_v1, e_v2]
    gsem = [gs0, gs1, gs2]
    wsem = [ws0, ws1, ws2]
    dsem = [ds0, ds1, ds2]

    pltpu.sync_copy(b_hbm, b_v)
    pltpu.sync_copy(a_hbm, a_v)
    pltpu.sync_copy(src_hbm.at[wid], src_all)
    pltpu.sync_copy(dst_hbm.at[wid], dst_all)

    b_ch = [b_v[pl.ds(j * 16, 16)] for j in range(D // 16)]
    a_ch = [a_v[pl.ds(j * 16, 16)] for j in range(D // 16)]
    lane = lax.iota(_i32, 16)
    bfly = [lane ^ m for m in (8, 4, 2, 1)]

    @pl.loop(0, TPN // 16)
    def _z(i):
        zb[pl.ds(i * 16, 16)] = jnp.zeros((16,), _f32)

    pltpu.sync_copy(zb, den_sh.at[pl.ds(sid * TPN, TPN)])
    plsc.subcore_barrier()

    def fire_g(i, k):
        pltpu.async_copy(xs_hbm.at[src_all.at[i]], xsr[k], gsem[k])
        pltpu.async_copy(xd_hbm.at[dst_all.at[i]], xdr[k], gsem[k])

    def wait_g(i, k):
        pltpu.make_async_copy(xs_hbm.at[src_all.at[i]], xsr[k], gsem[k]).wait()
        pltpu.make_async_copy(xd_hbm.at[dst_all.at[i]], xdr[k], gsem[k]).wait()

    def fire_w(i, k):
        off = base + i * B
        pltpu.async_copy(xsr[k], h_out.at[pl.ds(off, B)], wsem[k])
        pltpu.async_copy(ev[k], e_out.at[pl.ds(off, B)], wsem[k])

    def wait_w(i, k):
        off = base + i * B
        pltpu.make_async_copy(xsr[k], h_out.at[pl.ds(off, B)], wsem[k]).wait()
        pltpu.make_async_copy(ev[k], e_out.at[pl.ds(off, B)], wsem[k]).wait()

    def fire_d(i, k):
        pltpu.async_copy(ev[k], den_sh.at[dst_all.at[i]], dsem[k], add=True)

    def wait_d(i, k):
        pltpu.make_async_copy(ev[k], den_sh.at[dst_all.at[i]], dsem[k]).wait()

    def compute(i, k):
        xr, dr, e_b = xsr[k], xdr[k], ev[k]

        @pl.loop(0, B // 16)
        def _grp(t):
            sv = jnp.zeros((16,), _f32)
            for kk in range(16):
                e = t * 16 + kk
                acc = jnp.zeros((16,), _f32)
                for j in range(D // 16):
                    sl = pl.ds(j * 16, 16)
                    g = xr[e, sl] + dr[e, sl] + b_ch[j]
                    hv = jnp.maximum(g, 0.2 * g)
                    xr[e, sl] = hv
                    acc = acc + hv * a_ch[j]
                for p_ in bfly:
                    acc = acc + jnp.take_along_axis(acc, p_, axis=0)
                sv = jnp.where(lane == kk, acc, sv)
            e_b[pl.ds(t * 16, 16)] = jnp.exp(sv)

    fire_g(0, 0)
    fire_g(1, 1)

    @pl.loop(0, PITER)
    def _pipe(p):
        i0 = 3 * p
        wait_g(i0, 0)
        compute(i0, 0)
        fire_d(i0, 0)
        fire_w(i0, 0)

        @pl.when(p > 0)
        def _():
            wait_w(i0 - 1, 2)
            wait_d(i0 - 1, 2)

        @pl.when(p < PITER - 1)
        def _():
            fire_g(i0 + 2, 2)

        i1 = 3 * p + 1
        wait_g(i1, 1)
        compute(i1, 1)
        fire_d(i1, 1)
        fire_w(i1, 1)
        wait_w(i1 - 1, 0)
        wait_d(i1 - 1, 0)

        @pl.when(p < PITER - 1)
        def _():
            fire_g(i1 + 2, 0)

        @pl.when(p < PITER - 1)
        def _():
            i2 = 3 * p + 2
            wait_g(i2, 2)
            compute(i2, 2)
            fire_d(i2, 2)
            fire_w(i2, 2)
            wait_w(i2 - 1, 1)
            wait_d(i2 - 1, 1)
            fire_g(i2 + 2, 1)

    wait_w(BLOCKS - 1, 1)
    wait_d(BLOCKS - 1, 1)
    plsc.subcore_barrier()
    pltpu.sync_copy(den_sh.at[pl.ds(sid * TPN, TPN)],
                    den_out.at[cid, pl.ds(sid * TPN, TPN)])


def _k2b_body(den_ref, out_ref):
    d = den_ref[...]
    out_ref[...] = 1.0 / (d[0:1, :] + d[1:2, :] + 1e-16)


_k2b = pl.pallas_call(
    _k2b_body,
    in_specs=[pl.BlockSpec((NC, NPAD), lambda: (0, 0))],
    out_specs=pl.BlockSpec((1, NPAD), lambda: (0, 0)),
    out_shape=jax.ShapeDtypeStruct((1, NPAD), _f32),
)


@functools.partial(
    pl.kernel,
    out_type=jax.ShapeDtypeStruct((NC, AGGP, D), _f32),
    mesh=_mesh,
    scratch_types=[
        pltpu.VMEM((BLOCKS, B), _i32),
        pltpu.VMEM((B, D), _f32),
        pltpu.VMEM((B, D), _f32),
        pltpu.VMEM((B, D), _f32),
        pltpu.VMEM((B,), _i32),
        pltpu.VMEM((B,), _i32),
        pltpu.VMEM((B,), _i32),
        pltpu.VMEM((B,), _f32),
        pltpu.VMEM((B,), _f32),
        pltpu.VMEM((B,), _f32),
        pltpu.VMEM((B,), _f32),
        pltpu.VMEM((B,), _f32),
        pltpu.VMEM((B,), _f32),
        pltpu.VMEM_SHARED((AGGP, D), _f32),
        pltpu.SemaphoreType.DMA,
        pltpu.SemaphoreType.DMA,
        pltpu.SemaphoreType.DMA,
        pltpu.SemaphoreType.DMA,
        pltpu.SemaphoreType.DMA,
        pltpu.SemaphoreType.DMA,
    ],
)
def _k3(h_hbm, e_hbm, dst_hbm, hh_hbm, dinv_hbm, z_hbm,
        agg_out,
        dst_all, h_r0, h_r1, h_r2, hh_0, hh_1, hh_2,
        e_0, e_1, e_2, d_0, d_1, d_2, agg_sh,
        gs0, gs1, gs2, ss0, ss1, ss2):
    cid = lax.axis_index("c")
    sid = lax.axis_index("s")
    wid = sid * NC + cid
    base = wid * EPW

    hr = [h_r0, h_r1, h_r2]
    hhb = [hh_0, hh_1, hh_2]
    eb = [e_0, e_1, e_2]
    db = [d_0, d_1, d_2]
    gsem = [gs0, gs1, gs2]
    ssem = [ss0, ss1, ss2]

    pltpu.sync_copy(dst_hbm.at[wid], dst_all)
    pltpu.sync_copy(z_hbm.at[pl.ds(0, TP3)], agg_sh.at[pl.ds(sid * TP3, TP3)])
    plsc.subcore_barrier()

    def fire_g(i, k):
        off = base + i * B
        pltpu.async_copy(h_hbm.at[pl.ds(off, B)], hr[k], gsem[k])
        pltpu.async_copy(e_hbm.at[wid, i], eb[k], gsem[k])
        pltpu.async_copy(hh_hbm.at[wid, i], hhb[k], gsem[k])
        pltpu.async_copy(dinv_hbm.at[dst_all.at[i]], db[k], gsem[k])

    def wait_g(i, k):
        off = base + i * B
        pltpu.make_async_copy(h_hbm.at[pl.ds(off, B)], hr[k], gsem[k]).wait()
        pltpu.make_async_copy(e_hbm.at[wid, i], eb[k], gsem[k]).wait()
        pltpu.make_async_copy(hh_hbm.at[wid, i], hhb[k], gsem[k]).wait()
        pltpu.make_async_copy(dinv_hbm.at[dst_all.at[i]], db[k], gsem[k]).wait()

    def fire_s(i, k):
        pltpu.async_copy(hr[k], agg_sh.at[hhb[k]], ssem[k], add=True)

    def wait_s(i, k):
        pltpu.make_async_copy(hr[k], agg_sh.at[hhb[k]], ssem[k]).wait()

    def compute(i, k):
        hb, e_b, d_b = hr[k], eb[k], db[k]

        @pl.loop(0, B // 16)
        def _grp(t):
            sl16 = pl.ds(t * 16, 16)
            al16 = e_b[sl16] * d_b[sl16]
            for kk in range(16):
                e = t * 16 + kk
                al = al16[kk]
                for j in range(D // 16):
                    sl = pl.ds(j * 16, 16)
                    hb[e, sl] = hb[e, sl] * al

    fire_g(0, 0)
    fire_g(1, 1)

    @pl.loop(0, PITER)
    def _pipe(p):
        i0 = 3 * p
        wait_g(i0, 0)
        compute(i0, 0)
        fire_s(i0, 0)

        @pl.when(p > 0)
        def _():
            wait_s(i0 - 1, 2)

        @pl.when(p < PITER - 1)
        def _():
            fire_g(i0 + 2, 2)

        i1 = 3 * p + 1
        wait_g(i1, 1)
        compute(i1, 1)
        fire_s(i1, 1)
        wait_s(i1 - 1, 0)

        @pl.when(p < PITER - 1)
        def _():
            fire_g(i1 + 2, 0)

        @pl.when(p < PITER - 1)
        def _():
            i2 = 3 * p + 2
            wait_g(i2, 2)
            compute(i2, 2)
            fire_s(i2, 2)
            wait_s(i2 - 1, 1)
            fire_g(i2 + 2, 1)

    wait_s(BLOCKS - 1, 1)
    plsc.subcore_barrier()
    pltpu.sync_copy(agg_sh.at[pl.ds(sid * TP3, TP3)],
                    agg_out.at[cid, pl.ds(sid * TP3, TP3)])


def _k4_body(agg_ref, we_ref, be_ref, wo_ref, out_ref):
    av = agg_ref[...]
    a = av[0] + av[1]
    t = jnp.dot(a, we_ref[...], preferred_element_type=_f32) + be_ref[...]
    t = jnp.maximum(t, 0.2 * t)
    out_ref[...] = jnp.dot(t, wo_ref[...], preferred_element_type=_f32)


_k4 = pl.pallas_call(
    _k4_body,
    grid=(10,),
    in_specs=[
        pl.BlockSpec((2, N // 10, D), lambda i: (0, i, 0)),
        pl.BlockSpec((D, D), lambda i: (0, 0)),
        pl.BlockSpec((1, D), lambda i: (0, 0)),
        pl.BlockSpec((D, D), lambda i: (0, 0)),
    ],
    out_specs=pl.BlockSpec((N // 10, D), lambda i: (i, 0)),
    out_shape=jax.ShapeDtypeStruct((N, D), _f32),
)


def kernel(x, edge_index, H, W_src, W_dst, b_tsa, a_vec, W_etn, b_etn, W_out):
    src2 = edge_index[0].astype(_i32).reshape(NW, BLOCKS, B)
    dst2 = edge_index[1].astype(_i32).reshape(NW, BLOCKS, B)
    hh2 = H.astype(_i32).reshape(NW, BLOCKS, B)
    xs, xd = _k1(x, W_src, W_dst)
    h, ev, den = _k2(xs, xd, src2, dst2, b_tsa, a_vec)
    dinv = _k2b(den).reshape(NPAD)
    zeros = jnp.zeros((TPN, D), _f32)
    agg2 = _k3(h, ev.reshape(NW, BLOCKS, B), dst2, hh2, dinv, zeros)
    return _k4(agg2, W_etn, b_etn.reshape(1, D), W_out)
